# baseline (speedup 1.0000x reference)
import functools

import jax
import jax.numpy as jnp
from jax import lax
from jax.experimental import pallas as pl
from jax.experimental.pallas import tpu as pltpu
from jax.experimental.pallas import tpu_sc as plsc

HIDDEN = 16
EDIM = HIDDEN * 4
PAIR = 2 * EDIM
NC, NS, L = 2, 16, 16
NW = NC * NS
CHUNK = 256


def _relnorm_body(r0, r1, r2, r3, y0, y1, y2, y3):
    a, b, c, d = r0[...], r1[...], r2[...], r3[...]
    n2 = a * a + b * b + c * c + d * d
    rinv = jnp.minimum(lax.rsqrt(n2), 1e12)
    y0[...] = a * rinv
    y1[...] = b * rinv
    y2[...] = c * rinv
    y3[...] = d * rinv


def _normalize_relation_planar(relation_embedding):
    nrel = relation_embedding.shape[0]
    r = relation_embedding.reshape(nrel, HIDDEN, 4)
    comps = [r[:, :, c] for c in range(4)]
    outs = pl.pallas_call(
        _relnorm_body,
        out_shape=[jax.ShapeDtypeStruct((nrel, HIDDEN), jnp.float32)] * 4,
    )(*comps)
    return jnp.concatenate(outs, axis=1)


def _make_sc_scorer(batch):
    bpw = batch // NW
    nchunks = bpw // CHUNK
    ngroups = CHUNK // L

    mesh = plsc.VectorSubcoreMesh(core_axis_name="c", subcore_axis_name="s")

    @functools.partial(
        pl.kernel,
        out_type=jax.ShapeDtypeStruct((batch,), jnp.float32),
        mesh=mesh,
        compiler_params=pltpu.CompilerParams(needs_layout_passes=False),
        scratch_types=[
            pltpu.VMEM((CHUNK,), jnp.int32),
            pltpu.VMEM((CHUNK,), jnp.int32),
            pltpu.VMEM((CHUNK,), jnp.int32),
            pltpu.VMEM((CHUNK,), jnp.int32),
            pltpu.VMEM((CHUNK,), jnp.int32),
            pltpu.VMEM((CHUNK,), jnp.int32),
            pltpu.VMEM((CHUNK, PAIR), jnp.float32),
            pltpu.VMEM((CHUNK, PAIR), jnp.float32),
            pltpu.VMEM((CHUNK, PAIR), jnp.float32),
            pltpu.VMEM((bpw,), jnp.float32),
            pltpu.SemaphoreType.DMA,
        ],
    )
    def scorer(hrow_hbm, hoff_hbm, rrow_hbm, roff_hbm, trow_hbm, toff_hbm,
               ent_hbm, reln_hbm, phi_hbm,
               row_h, row_r, row_t, off_h, off_r, off_t,
               rows_h, rows_r, rows_t, phi_v, sem):
        wid = lax.axis_index("s") * NC + lax.axis_index("c")
        base = wid * bpw

        iota = lax.broadcasted_iota(jnp.int32, (L,), 0)

        def chunk_body(c, carry):
            cbase = base + c * CHUNK
            pltpu.sync_copy(hrow_hbm.at[pl.ds(cbase, CHUNK)], row_h)
            pltpu.sync_copy(rrow_hbm.at[pl.ds(cbase, CHUNK)], row_r)
            pltpu.sync_copy(trow_hbm.at[pl.ds(cbase, CHUNK)], row_t)
            pltpu.sync_copy(hoff_hbm.at[pl.ds(cbase, CHUNK)], off_h)
            pltpu.sync_copy(roff_hbm.at[pl.ds(cbase, CHUNK)], off_r)
            pltpu.sync_copy(toff_hbm.at[pl.ds(cbase, CHUNK)], off_t)
            ch = pltpu.async_copy(ent_hbm.at[row_h], rows_h, sem)
            cr = pltpu.async_copy(reln_hbm.at[row_r], rows_r, sem)
            ct = pltpu.async_copy(ent_hbm.at[row_t], rows_t, sem)
            ch.wait()
            cr.wait()
            ct.wait()

            def group_body(g, carry2):
                rowloc = g * L + iota
                h64 = off_h[pl.ds(g * L, L)]
                r64 = off_r[pl.ds(g * L, L)]
                t64 = off_t[pl.ds(g * L, L)]
                acc = jnp.zeros((L,), jnp.float32)
                for k in range(HIDDEN):
                    hw = plsc.load_gather(rows_h, [rowloc, h64 + (4 * k + 0)])
                    hx = plsc.load_gather(rows_h, [rowloc, h64 + (4 * k + 1)])
                    hy = plsc.load_gather(rows_h, [rowloc, h64 + (4 * k + 2)])
                    hz = plsc.load_gather(rows_h, [rowloc, h64 + (4 * k + 3)])
                    tw = plsc.load_gather(rows_t, [rowloc, t64 + (4 * k + 0)])
                    tx = plsc.load_gather(rows_t, [rowloc, t64 + (4 * k + 1)])
                    ty = plsc.load_gather(rows_t, [rowloc, t64 + (4 * k + 2)])
                    tz = plsc.load_gather(rows_t, [rowloc, t64 + (4 * k + 3)])
                    rw = plsc.load_gather(rows_r, [rowloc, r64 + k])
                    rx = plsc.load_gather(rows_r, [rowloc, r64 + (HIDDEN + k)])
                    ry = plsc.load_gather(rows_r, [rowloc, r64 + (2 * HIDDEN + k)])
                    rz = plsc.load_gather(rows_r, [rowloc, r64 + (3 * HIDDEN + k)])

                    p = hw * rw - hx * rx - hy * ry - hz * rz
                    q = hw * rx + hx * rw + hy * rz - hz * ry
                    u = hw * ry - hx * rz + hy * rw + hz * rx
                    v = hw * rz + hx * ry - hy * rx + hz * rw
                    acc = acc + (p * tw + q * tx + u * ty + v * tz)
                phi_v[pl.ds(c * CHUNK + g * L, L)] = acc
                return carry2

            lax.fori_loop(0, ngroups, group_body, 0)
            return carry

        lax.fori_loop(0, nchunks, chunk_body, 0)
        pltpu.sync_copy(phi_v, phi_hbm.at[pl.ds(base, bpw)])

    return scorer


def _loss_body(phi, y, out):
    out[...] = jnp.log(1.0 + jnp.exp(-y[...] * phi[...]))


def _loss(phi, Y):
    batch = phi.shape[0]
    r = batch // 128
    out = pl.pallas_call(
        _loss_body,
        out_shape=jax.ShapeDtypeStruct((r, 128), jnp.float32),
    )(phi.reshape(r, 128), Y.reshape(r, 128))
    return out.reshape(batch)


@jax.jit
def kernel(sample, Y, entity_embedding, relation_embedding):
    batch = sample.shape[0]
    nentity = entity_embedding.shape[0]

    reln_planar = _normalize_relation_planar(relation_embedding)
    reln_pair = reln_planar.reshape(-1, PAIR)

    ent_pair = entity_embedding.reshape(nentity // 2, PAIR)

    heads = sample[:, 0].astype(jnp.int32)
    rels = sample[:, 1].astype(jnp.int32)
    tails = sample[:, 2].astype(jnp.int32)

    scorer = _make_sc_scorer(batch)
    phi = scorer(
        heads >> 1, (heads & 1) * EDIM,
        rels >> 1, (rels & 1) * EDIM,
        tails >> 1, (tails & 1) * EDIM,
        ent_pair, reln_pair,
    )

    loss = _loss(phi, Y)
    return (loss, Y)


# speedup vs baseline: 1.0511x; 1.0511x over previous
import functools

import jax
import jax.numpy as jnp
import numpy as np
from jax import lax
from jax.experimental import pallas as pl
from jax.experimental.pallas import tpu as pltpu
from jax.experimental.pallas import tpu_sc as plsc

HIDDEN = 16
EDIM = HIDDEN * 4
PAIR = 2 * EDIM
NC, NS, L = 2, 16, 16
NW = NC * NS
CHUNK = 256
BUFW = L + 1


def _relnorm_body(r, m, y):
    x = r[...]
    n2 = jax.lax.dot(x * x, m[...], precision=jax.lax.Precision.HIGHEST)
    rinv = jnp.minimum(lax.rsqrt(n2), 1e12)
    y[...] = x * jax.lax.dot(rinv, m[...].T, precision=jax.lax.Precision.HIGHEST)


def _normalize_relation(relation_embedding):
    m = jnp.asarray(np.kron(np.eye(HIDDEN, dtype=np.float32),
                            np.ones((4, 1), dtype=np.float32)))
    return pl.pallas_call(
        _relnorm_body,
        out_shape=jax.ShapeDtypeStruct(relation_embedding.shape, jnp.float32),
    )(relation_embedding, m)


def _take(x, perm):
    return x.at[perm].get(mode="promise_in_bounds")


def _make_sc_scorer(batch):
    bpw = batch // NW
    nchunks = bpw // CHUNK
    ngroups = CHUNK // L

    mesh = plsc.VectorSubcoreMesh(core_axis_name="c", subcore_axis_name="s")

    @functools.partial(
        pl.kernel,
        out_type=jax.ShapeDtypeStruct((batch,), jnp.float32),
        mesh=mesh,
        compiler_params=pltpu.CompilerParams(needs_layout_passes=False),
        scratch_types=[
            pltpu.VMEM((CHUNK,), jnp.int32),
            pltpu.VMEM((CHUNK,), jnp.int32),
            pltpu.VMEM((CHUNK,), jnp.int32),
            pltpu.VMEM((CHUNK,), jnp.int32),
            pltpu.VMEM((CHUNK,), jnp.int32),
            pltpu.VMEM((CHUNK,), jnp.int32),
            pltpu.VMEM((CHUNK, PAIR), jnp.float32),
            pltpu.VMEM((CHUNK, PAIR), jnp.float32),
            pltpu.VMEM((CHUNK, PAIR), jnp.float32),
            pltpu.VMEM((L * BUFW,), jnp.float32),
            pltpu.VMEM((bpw,), jnp.float32),
            pltpu.SemaphoreType.DMA,
        ],
    )
    def scorer(hrow_hbm, hoff_hbm, rrow_hbm, roff_hbm, trow_hbm, toff_hbm,
               ent_hbm, reln_hbm, phi_hbm,
               row_h, row_r, row_t, off_h, off_r, off_t,
               rows_h, rows_r, rows_t, buf, phi_v, sem):
        wid = lax.axis_index("s") * NC + lax.axis_index("c")
        base = wid * bpw

        iota = lax.broadcasted_iota(jnp.int32, (L,), 0)
        comp = iota & 3
        perm_a = [(iota & ~3) | j for j in range(4)]
        perm_b = [iota ^ j for j in range(4)]
        one = jnp.full((L,), 1.0, jnp.float32)
        neg = jnp.full((L,), -1.0, jnp.float32)
        sign_b = [
            None,
            jnp.where((comp & 1) != 0, one, neg),
            jnp.where((comp == 1) | (comp == 2), one, neg),
            jnp.where(comp >= 2, one, neg),
        ]

        def chunk_body(c, carry):
            cbase = base + c * CHUNK
            pltpu.sync_copy(hrow_hbm.at[pl.ds(cbase, CHUNK)], row_h)
            pltpu.sync_copy(rrow_hbm.at[pl.ds(cbase, CHUNK)], row_r)
            pltpu.sync_copy(trow_hbm.at[pl.ds(cbase, CHUNK)], row_t)
            pltpu.sync_copy(hoff_hbm.at[pl.ds(cbase, CHUNK)], off_h)
            pltpu.sync_copy(roff_hbm.at[pl.ds(cbase, CHUNK)], off_r)
            pltpu.sync_copy(toff_hbm.at[pl.ds(cbase, CHUNK)], off_t)
            ch = pltpu.async_copy(ent_hbm.at[row_h], rows_h, sem)
            cr = pltpu.async_copy(reln_hbm.at[row_r], rows_r, sem)
            ct = pltpu.async_copy(ent_hbm.at[row_t], rows_t, sem)
            ch.wait()
            cr.wait()
            ct.wait()

            def group_body(g, carry2):
                gbase = g * L
                hoffv = off_h[pl.ds(gbase, L)]
                roffv = off_r[pl.ds(gbase, L)]
                toffv = off_t[pl.ds(gbase, L)]
                for j in range(L):
                    s = gbase + j
                    hoff = hoffv[j]
                    roff = roffv[j]
                    toff = toffv[j]
                    acc = jnp.zeros((L,), jnp.float32)
                    for q in range(4):
                        a = rows_h[s, pl.ds(hoff + 16 * q, L)]
                        b = rows_r[s, pl.ds(roff + 16 * q, L)]
                        t = rows_t[s, pl.ds(toff + 16 * q, L)]
                        h = _take(a, perm_a[0]) * b
                        for jj in range(1, 4):
                            bv = _take(b, perm_b[jj]) * sign_b[jj]
                            h = h + _take(a, perm_a[jj]) * bv
                        acc = acc + h * t
                    buf[pl.ds(j * BUFW, L)] = acc
                p = plsc.load_gather(buf, [iota * BUFW])
                for k in range(1, L):
                    p = p + plsc.load_gather(buf, [iota * BUFW + k])
                phi_v[pl.ds(c * CHUNK + gbase, L)] = p
                return carry2

            lax.fori_loop(0, ngroups, group_body, 0)
            return carry

        lax.fori_loop(0, nchunks, chunk_body, 0)
        pltpu.sync_copy(phi_v, phi_hbm.at[pl.ds(base, bpw)])

    return scorer


def _loss_body(phi, y, out):
    out[...] = jnp.log(1.0 + jnp.exp(-y[...] * phi[...]))


def _loss(phi, Y):
    batch = phi.shape[0]
    r = batch // 128
    out = pl.pallas_call(
        _loss_body,
        out_shape=jax.ShapeDtypeStruct((r, 128), jnp.float32),
    )(phi.reshape(r, 128), Y.reshape(r, 128))
    return out.reshape(batch)


@jax.jit
def kernel(sample, Y, entity_embedding, relation_embedding):
    batch = sample.shape[0]
    nentity = entity_embedding.shape[0]

    reln = _normalize_relation(relation_embedding)
    reln_pair = reln.reshape(-1, PAIR)

    ent_pair = entity_embedding.reshape(nentity // 2, PAIR)

    heads = sample[:, 0].astype(jnp.int32)
    rels = sample[:, 1].astype(jnp.int32)
    tails = sample[:, 2].astype(jnp.int32)

    scorer = _make_sc_scorer(batch)
    phi = scorer(
        heads >> 1, (heads & 1) * EDIM,
        rels >> 1, (rels & 1) * EDIM,
        tails >> 1, (tails & 1) * EDIM,
        ent_pair, reln_pair,
    )

    loss = _loss(phi, Y)
    return (loss, Y)


# speedup vs baseline: 1.7212x; 1.6376x over previous
import functools

import jax
import jax.numpy as jnp
import numpy as np
from jax import lax
from jax.experimental import pallas as pl
from jax.experimental.pallas import tpu as pltpu
from jax.experimental.pallas import tpu_sc as plsc

HIDDEN = 16
EDIM = HIDDEN * 4
PAIR = 2 * EDIM
NC, NS, L = 2, 16, 16
NW = NC * NS
CHUNK = 256
BUFW = L + 1


def _relnorm_body(r, m, y):
    x = r[...]
    n2 = jax.lax.dot(x * x, m[...], precision=jax.lax.Precision.HIGHEST)
    rinv = jnp.minimum(lax.rsqrt(n2), 1e12)
    y[...] = x * jax.lax.dot(rinv, m[...].T, precision=jax.lax.Precision.HIGHEST)


def _normalize_relation(relation_embedding):
    m = jnp.asarray(np.kron(np.eye(HIDDEN, dtype=np.float32),
                            np.ones((4, 1), dtype=np.float32)))
    return pl.pallas_call(
        _relnorm_body,
        out_shape=jax.ShapeDtypeStruct(relation_embedding.shape, jnp.float32),
    )(relation_embedding, m)


def _take(x, perm):
    return x.at[perm].get(mode="promise_in_bounds")


def _make_sc_scorer(batch):
    bpw = batch // NW
    nchunks = bpw // CHUNK
    ngroups = CHUNK // L

    mesh = plsc.VectorSubcoreMesh(core_axis_name="c", subcore_axis_name="s")

    @functools.partial(
        pl.kernel,
        out_type=jax.ShapeDtypeStruct((batch,), jnp.float32),
        mesh=mesh,
        compiler_params=pltpu.CompilerParams(needs_layout_passes=False),
        scratch_types=[
            pltpu.VMEM((CHUNK,), jnp.int32),
            pltpu.VMEM((CHUNK,), jnp.int32),
            pltpu.VMEM((CHUNK,), jnp.int32),
            pltpu.VMEM((CHUNK,), jnp.int32),
            pltpu.VMEM((CHUNK, EDIM), jnp.float32),
            pltpu.VMEM((CHUNK, PAIR), jnp.float32),
            pltpu.VMEM((CHUNK, EDIM), jnp.float32),
            pltpu.VMEM((L * BUFW,), jnp.float32),
            pltpu.VMEM((bpw,), jnp.float32),
            pltpu.SemaphoreType.DMA,
            pltpu.SemaphoreType.DMA,
        ],
    )
    def scorer(heads_hbm, rrow_hbm, roff_hbm, tails_hbm,
               ent_hbm, reln_hbm, phi_hbm,
               row_h, row_r, row_t, off_r,
               rows_h, rows_r, rows_t, buf, phi_v, sem, rsem):
        wid = lax.axis_index("s") * NC + lax.axis_index("c")
        base = wid * bpw

        iota = lax.broadcasted_iota(jnp.int32, (L,), 0)
        comp = iota & 3
        perm_a = [(iota & ~3) | j for j in range(4)]
        perm_b = [iota ^ j for j in range(4)]
        one = jnp.full((L,), 1.0, jnp.float32)
        neg = jnp.full((L,), -1.0, jnp.float32)
        sign_b = [
            None,
            jnp.where((comp & 1) != 0, one, neg),
            jnp.where((comp == 1) | (comp == 2), one, neg),
            jnp.where(comp >= 2, one, neg),
        ]

        def chunk_body(c, carry):
            cbase = base + c * CHUNK
            pltpu.sync_copy(heads_hbm.at[pl.ds(cbase, CHUNK)], row_h)
            pltpu.sync_copy(rrow_hbm.at[pl.ds(cbase, CHUNK)], row_r)
            pltpu.sync_copy(tails_hbm.at[pl.ds(cbase, CHUNK)], row_t)
            pltpu.sync_copy(roff_hbm.at[pl.ds(cbase, CHUNK)], off_r)
            cr = pltpu.async_copy(reln_hbm.at[row_r], rows_r, rsem)

            def issue_group(g, carry2):
                gbase = g * L
                hv = row_h[pl.ds(gbase, L)]
                tv = row_t[pl.ds(gbase, L)]
                for j in range(L):
                    pltpu.async_copy(
                        ent_hbm.at[hv[j]], rows_h.at[gbase + j], sem)
                    pltpu.async_copy(
                        ent_hbm.at[tv[j]], rows_t.at[gbase + j], sem)
                return carry2

            lax.fori_loop(0, ngroups, issue_group, 0)
            pltpu.make_async_copy(
                ent_hbm.at[pl.ds(0, CHUNK)], rows_h, sem).wait()
            pltpu.make_async_copy(
                ent_hbm.at[pl.ds(0, CHUNK)], rows_t, sem).wait()
            cr.wait()

            def group_body(g, carry2):
                gbase = g * L
                roffv = off_r[pl.ds(gbase, L)]
                for j in range(L):
                    s = gbase + j
                    roff = roffv[j]
                    acc = jnp.zeros((L,), jnp.float32)
                    for q in range(4):
                        a = rows_h[s, pl.ds(16 * q, L)]
                        b = rows_r[s, pl.ds(roff + 16 * q, L)]
                        t = rows_t[s, pl.ds(16 * q, L)]
                        h = _take(a, perm_a[0]) * b
                        for jj in range(1, 4):
                            bv = _take(b, perm_b[jj]) * sign_b[jj]
                            h = h + _take(a, perm_a[jj]) * bv
                        acc = acc + h * t
                    buf[pl.ds(j * BUFW, L)] = acc
                p = plsc.load_gather(buf, [iota * BUFW])
                for k in range(1, L):
                    p = p + plsc.load_gather(buf, [iota * BUFW + k])
                phi_v[pl.ds(c * CHUNK + gbase, L)] = p
                return carry2

            lax.fori_loop(0, ngroups, group_body, 0)
            return carry

        lax.fori_loop(0, nchunks, chunk_body, 0)
        pltpu.sync_copy(phi_v, phi_hbm.at[pl.ds(base, bpw)])

    return scorer


def _loss_body(phi, y, out):
    out[...] = jnp.log(1.0 + jnp.exp(-y[...] * phi[...]))


def _loss(phi, Y):
    batch = phi.shape[0]
    r = batch // 128
    out = pl.pallas_call(
        _loss_body,
        out_shape=jax.ShapeDtypeStruct((r, 128), jnp.float32),
    )(phi.reshape(r, 128), Y.reshape(r, 128))
    return out.reshape(batch)


@jax.jit
def kernel(sample, Y, entity_embedding, relation_embedding):
    batch = sample.shape[0]

    reln = _normalize_relation(relation_embedding)
    reln_pair = reln.reshape(-1, PAIR)

    heads = sample[:, 0].astype(jnp.int32)
    rels = sample[:, 1].astype(jnp.int32)
    tails = sample[:, 2].astype(jnp.int32)

    scorer = _make_sc_scorer(batch)
    phi = scorer(
        heads,
        rels >> 1, (rels & 1) * EDIM,
        tails,
        entity_embedding, reln_pair,
    )

    loss = _loss(phi, Y)
    return (loss, Y)
